# baseline (speedup 1.0000x reference)
import functools

import jax
import jax.numpy as jnp
from jax import lax
from jax.experimental import pallas as pl
from jax.experimental.pallas import tpu as pltpu
from jax.experimental.pallas import tpu_sc as plsc

VOCAB = 1000000
VE_DIM = 64
MODEL_DIM = 128
B = 4096
L = 200
N = B * L

NC = 2
NS = 16
NW = NC * NS
PER_W = N // NW
CHUNK = 128
N_CHUNKS = PER_W // CHUNK
NBUF = 4


def _sc_gather_body(idx_hbm, table_hbm, out_hbm, idx_v, rows_v, gsems):
    c = lax.axis_index("c")
    s = lax.axis_index("s")
    wid = s * NC + c

    pltpu.sync_copy(idx_hbm.at[wid], idx_v)

    for b in range(NBUF):
        pltpu.async_copy(table_hbm.at[idx_v.at[b]], rows_v.at[b], gsems.at[b])

    out_base = wid * N_CHUNKS

    def outer(j0, carry):
        for b in range(NBUF):
            j = j0 * NBUF + b
            pltpu.make_async_copy(
                table_hbm.at[idx_v.at[j]], rows_v.at[b], gsems.at[b]
            ).wait()
            pltpu.sync_copy(rows_v.at[b], out_hbm.at[out_base + j])

            @pl.when(j + NBUF < N_CHUNKS)
            def _refill(b=b, j=j):
                pltpu.async_copy(
                    table_hbm.at[idx_v.at[j + NBUF]], rows_v.at[b], gsems.at[b]
                )

        return carry

    lax.fori_loop(0, N_CHUNKS // NBUF, outer, None)


@jax.jit
def _sc_gather(idx, table):
    mesh = plsc.VectorSubcoreMesh(core_axis_name="c", subcore_axis_name="s")
    return pl.kernel(
        _sc_gather_body,
        out_type=jax.ShapeDtypeStruct((NW * N_CHUNKS, CHUNK, VE_DIM), jnp.float32),
        mesh=mesh,
        scratch_types=[
            pltpu.VMEM((N_CHUNKS, CHUNK), jnp.int32),
            pltpu.VMEM((NBUF, CHUNK, VE_DIM), jnp.float32),
            pltpu.SemaphoreType.DMA((NBUF,)),
        ],
        compiler_params=pltpu.CompilerParams(use_tc_tiling_on_sc=False),
    )(idx, table)


def _mm_body(x_ref, w_ref, s_ref, o_ref):
    o_ref[...] = jnp.dot(
        x_ref[...], w_ref[...], preferred_element_type=jnp.float32
    ) * s_ref[0]


ROWS_BLK = 2048


@jax.jit
def _tc_project(x, w_t, scale):
    grid = (N // ROWS_BLK,)
    return pl.pallas_call(
        _mm_body,
        grid=grid,
        in_specs=[
            pl.BlockSpec((ROWS_BLK, VE_DIM), lambda i: (i, 0)),
            pl.BlockSpec((VE_DIM, MODEL_DIM), lambda i: (0, 0)),
            pl.BlockSpec(memory_space=pltpu.SMEM),
        ],
        out_specs=pl.BlockSpec((ROWS_BLK, MODEL_DIM), lambda i: (i, 0)),
        out_shape=jax.ShapeDtypeStruct((N, MODEL_DIM), jnp.float32),
    )(x, w_t, scale)


def kernel(token_ids, embed_weight, proj_weight, scale):
    idx = token_ids.astype(jnp.int32).reshape(NW, N_CHUNKS, CHUNK)
    gathered = _sc_gather(idx, embed_weight)
    gathered = gathered.reshape(N, VE_DIM)
    out = _tc_project(gathered, proj_weight.T, scale.reshape(1))
    return out.reshape(B, L, MODEL_DIM)


# speedup vs baseline: 1.1402x; 1.1402x over previous
import jax
import jax.numpy as jnp
from jax import lax
from jax.experimental import pallas as pl
from jax.experimental.pallas import tpu as pltpu
from jax.experimental.pallas import tpu_sc as plsc

VOCAB = 1000000
VE_DIM = 64
MODEL_DIM = 128
B = 4096
L = 200
N = B * L

NC = 2
NS = 16
NW = NC * NS
PER_W = N // NW
CHUNK = 128
HALF = CHUNK // 2
N_CHUNKS = PER_W // CHUNK
NBUF = 4

R_PAIR = 1024
N_BLOCKS = N // (2 * R_PAIR)


def _sc_gather_body(idx_hbm, table_hbm, out_hbm, idx_v, rows_v, gsems):
    c = lax.axis_index("c")
    s = lax.axis_index("s")
    wid = s * NC + c

    pltpu.sync_copy(idx_hbm.at[wid], idx_v)

    for b in range(NBUF):
        pltpu.async_copy(table_hbm.at[idx_v.at[b]], rows_v.at[b], gsems.at[b])

    out_base = wid * N_CHUNKS

    def outer(j0, carry):
        for b in range(NBUF):
            j = j0 * NBUF + b
            pltpu.make_async_copy(
                table_hbm.at[idx_v.at[j]], rows_v.at[b], gsems.at[b]
            ).wait()
            pltpu.sync_copy(rows_v.at[b], out_hbm.at[out_base + j])

            @pl.when(j + NBUF < N_CHUNKS)
            def _refill(b=b, j=j):
                pltpu.async_copy(
                    table_hbm.at[idx_v.at[j + NBUF]], rows_v.at[b], gsems.at[b]
                )

        return carry

    lax.fori_loop(0, N_CHUNKS // NBUF, outer, None)


def _sc_gather(idx, table):
    mesh = plsc.VectorSubcoreMesh(core_axis_name="c", subcore_axis_name="s")
    return pl.kernel(
        _sc_gather_body,
        out_type=jax.ShapeDtypeStruct((NW * N_CHUNKS, CHUNK, VE_DIM), jnp.float32),
        mesh=mesh,
        scratch_types=[
            pltpu.VMEM((N_CHUNKS, CHUNK), jnp.int32),
            pltpu.VMEM((NBUF, CHUNK, VE_DIM), jnp.float32),
            pltpu.SemaphoreType.DMA((NBUF,)),
        ],
        compiler_params=pltpu.CompilerParams(use_tc_tiling_on_sc=False),
    )(idx, table)


def _mm_body(x_ref, w_ref, s_ref, o_ref):
    sc = s_ref[0]
    w = w_ref[...]
    o_ref[:R_PAIR, :] = (
        jnp.dot(x_ref[:, :VE_DIM], w, preferred_element_type=jnp.float32) * sc
    )
    o_ref[R_PAIR:, :] = (
        jnp.dot(x_ref[:, VE_DIM:], w, preferred_element_type=jnp.float32) * sc
    )


def _tc_project(x, w_t, scale):
    return pl.pallas_call(
        _mm_body,
        grid=(N_BLOCKS,),
        in_specs=[
            pl.BlockSpec((R_PAIR, 2 * VE_DIM), lambda i: (i, 0)),
            pl.BlockSpec((VE_DIM, MODEL_DIM), lambda i: (0, 0)),
            pl.BlockSpec(memory_space=pltpu.SMEM),
        ],
        out_specs=pl.BlockSpec((2 * R_PAIR, MODEL_DIM), lambda i: (i, 0)),
        out_shape=jax.ShapeDtypeStruct((N, MODEL_DIM), jnp.float32),
    )(x, w_t, scale)


def kernel(token_ids, embed_weight, proj_weight, scale):
    ids = token_ids.astype(jnp.int32).reshape(-1)
    idx_sc = (
        ids.reshape(-1, 2, R_PAIR)
        .transpose(0, 2, 1)
        .reshape(NW, N_CHUNKS, CHUNK)
    )
    gathered = _sc_gather(idx_sc, embed_weight)
    paired = gathered.reshape(N // 2, 2 * VE_DIM)
    out = _tc_project(paired, proj_weight.T, scale.reshape(1))
    return out.reshape(B, L, MODEL_DIM)


# speedup vs baseline: 1.3110x; 1.1498x over previous
import jax
import jax.numpy as jnp
from jax import lax
from jax.experimental import pallas as pl
from jax.experimental.pallas import tpu as pltpu
from jax.experimental.pallas import tpu_sc as plsc

VOCAB = 1000000
VE_DIM = 64
MODEL_DIM = 128
B = 4096
L = 200
N = B * L

NC = 2
NS = 16
NW = NC * NS
PER_W = N // NW
CHUNK = 128
HALF = CHUNK // 2
N_CHUNKS = PER_W // CHUNK
NBUF = 4

R_PAIR = 4096
N_BLOCKS = N // (2 * R_PAIR)


def _sc_gather_body(idx_hbm, table_hbm, out_hbm, idx_v, rows_v, gsems):
    c = lax.axis_index("c")
    s = lax.axis_index("s")
    wid = s * NC + c

    pltpu.sync_copy(idx_hbm.at[wid], idx_v)

    for b in range(NBUF):
        pltpu.async_copy(table_hbm.at[idx_v.at[b]], rows_v.at[b], gsems.at[b])

    out_base = wid * N_CHUNKS

    def outer(j0, carry):
        for b in range(NBUF):
            j = j0 * NBUF + b
            pltpu.make_async_copy(
                table_hbm.at[idx_v.at[j]], rows_v.at[b], gsems.at[b]
            ).wait()
            pltpu.sync_copy(rows_v.at[b], out_hbm.at[out_base + j])

            @pl.when(j + NBUF < N_CHUNKS)
            def _refill(b=b, j=j):
                pltpu.async_copy(
                    table_hbm.at[idx_v.at[j + NBUF]], rows_v.at[b], gsems.at[b]
                )

        return carry

    lax.fori_loop(0, N_CHUNKS // NBUF, outer, None)


def _sc_gather(idx, table):
    mesh = plsc.VectorSubcoreMesh(core_axis_name="c", subcore_axis_name="s")
    return pl.kernel(
        _sc_gather_body,
        out_type=jax.ShapeDtypeStruct((NW * N_CHUNKS, CHUNK, VE_DIM), jnp.float32),
        mesh=mesh,
        scratch_types=[
            pltpu.VMEM((N_CHUNKS, CHUNK), jnp.int32),
            pltpu.VMEM((NBUF, CHUNK, VE_DIM), jnp.float32),
            pltpu.SemaphoreType.DMA((NBUF,)),
        ],
        compiler_params=pltpu.CompilerParams(use_tc_tiling_on_sc=False),
    )(idx, table)


def _mm_body(x_ref, w_ref, s_ref, o_ref):
    sc = s_ref[0]
    w = w_ref[...]
    o_ref[:R_PAIR, :] = (
        jnp.dot(x_ref[:, :VE_DIM], w, preferred_element_type=jnp.float32) * sc
    )
    o_ref[R_PAIR:, :] = (
        jnp.dot(x_ref[:, VE_DIM:], w, preferred_element_type=jnp.float32) * sc
    )


def _tc_project(x, w_t, scale):
    return pl.pallas_call(
        _mm_body,
        grid=(N_BLOCKS,),
        in_specs=[
            pl.BlockSpec((R_PAIR, 2 * VE_DIM), lambda i: (i, 0)),
            pl.BlockSpec((VE_DIM, MODEL_DIM), lambda i: (0, 0)),
            pl.BlockSpec(memory_space=pltpu.SMEM),
        ],
        out_specs=pl.BlockSpec((2 * R_PAIR, MODEL_DIM), lambda i: (i, 0)),
        out_shape=jax.ShapeDtypeStruct((N, MODEL_DIM), jnp.float32),
    )(x, w_t, scale)


def kernel(token_ids, embed_weight, proj_weight, scale):
    ids = token_ids.astype(jnp.int32).reshape(-1)
    idx_sc = (
        ids.reshape(-1, 2, R_PAIR)
        .transpose(0, 2, 1)
        .reshape(NW, N_CHUNKS, CHUNK)
    )
    gathered = _sc_gather(idx_sc, embed_weight)
    paired = gathered.reshape(N // 2, 2 * VE_DIM)
    out = _tc_project(paired, proj_weight.T, scale.reshape(1))
    return out.reshape(B, L, MODEL_DIM)


# speedup vs baseline: 2.3296x; 1.7770x over previous
import jax
import jax.numpy as jnp
from jax import lax
from jax.experimental import pallas as pl
from jax.experimental.pallas import tpu as pltpu
from jax.experimental.pallas import tpu_sc as plsc

VOCAB = 1000000
VE_DIM = 64
MODEL_DIM = 128
B = 4096
L = 200
N = B * L
HN = N // 2

NC = 2
NS = 16
NW = NC * NS
SPAN = HN // NW
CHUNK = 128
N_CHUNKS = SPAN // CHUNK
NBUF = 4

TBLK = 8192
HBLK = TBLK // 2
T_GRID = (VOCAB + TBLK - 1) // TBLK
VPAD = T_GRID * TBLK

R_PAIR = 4096
N_BLOCKS = HN // R_PAIR


def _tr_body(x_ref, o_ref):
    o_ref[:, :VE_DIM] = x_ref[:, :HBLK].T
    o_ref[:, VE_DIM:] = x_ref[:, HBLK:].T


def _tc_transpose(wt):
    return pl.pallas_call(
        _tr_body,
        grid=(T_GRID,),
        in_specs=[pl.BlockSpec((VE_DIM, TBLK), lambda i: (0, i))],
        out_specs=pl.BlockSpec((HBLK, 2 * VE_DIM), lambda i: (i, 0)),
        out_shape=jax.ShapeDtypeStruct((T_GRID * HBLK, 2 * VE_DIM), jnp.float32),
    )(wt)


def _sc_gather_body(idx_hbm, table_hbm, out_hbm, idx_v, bufa, bufb, sema, semb):
    c = lax.axis_index("c")
    s = lax.axis_index("s")
    wid = s * NC + c

    pltpu.sync_copy(idx_hbm.at[pl.ds(wid * SPAN, SPAN)], idx_v.at[0])
    pltpu.sync_copy(idx_hbm.at[pl.ds(HN + wid * SPAN, SPAN)], idx_v.at[1])

    def _fill(j, b):
        pltpu.async_copy(
            table_hbm.at[idx_v.at[0, pl.ds(j * CHUNK, CHUNK)]], bufa.at[b],
            sema.at[b],
        )
        pltpu.async_copy(
            table_hbm.at[idx_v.at[1, pl.ds(j * CHUNK, CHUNK)]], bufb.at[b],
            semb.at[b],
        )

    for b in range(NBUF):
        _fill(b, b)

    row0 = wid * SPAN

    def outer(j0, carry):
        for b in range(NBUF):
            j = j0 * NBUF + b
            pltpu.make_async_copy(
                table_hbm.at[idx_v.at[0, pl.ds(j * CHUNK, CHUNK)]], bufa.at[b],
                sema.at[b],
            ).wait()
            pltpu.make_async_copy(
                table_hbm.at[idx_v.at[1, pl.ds(j * CHUNK, CHUNK)]], bufb.at[b],
                semb.at[b],
            ).wait()
            r = row0 + j * CHUNK
            pltpu.sync_copy(bufa.at[b],
                            out_hbm.at[pl.ds(r, CHUNK), pl.ds(0, VE_DIM)])
            pltpu.sync_copy(bufb.at[b],
                            out_hbm.at[pl.ds(r, CHUNK), pl.ds(VE_DIM, VE_DIM)])

            @pl.when(j + NBUF < N_CHUNKS)
            def _refill(b=b, j=j):
                _fill(j + NBUF, b)

        return carry

    lax.fori_loop(0, N_CHUNKS // NBUF, outer, None)


def _sc_gather(idx, table):
    mesh = plsc.VectorSubcoreMesh(core_axis_name="c", subcore_axis_name="s")
    return pl.kernel(
        _sc_gather_body,
        out_type=jax.ShapeDtypeStruct((HN, 2 * VE_DIM), jnp.float32),
        name="sc_pair_gather",
        mesh=mesh,
        scratch_types=[
            pltpu.VMEM((2, SPAN), jnp.int32),
            pltpu.VMEM((NBUF, CHUNK, VE_DIM), jnp.float32),
            pltpu.VMEM((NBUF, CHUNK, VE_DIM), jnp.float32),
            pltpu.SemaphoreType.DMA((NBUF,)),
            pltpu.SemaphoreType.DMA((NBUF,)),
        ],
        compiler_params=pltpu.CompilerParams(use_tc_tiling_on_sc=False),
    )(idx, table)


def _mm_body(x_ref, w_ref, s_ref, o_ref):
    sc = s_ref[0]
    w = w_ref[...]
    o_ref[0] = (
        jnp.dot(x_ref[:, :VE_DIM], w, preferred_element_type=jnp.float32) * sc
    )
    o_ref[1] = (
        jnp.dot(x_ref[:, VE_DIM:], w, preferred_element_type=jnp.float32) * sc
    )


def _tc_project(x, w_t, scale):
    return pl.pallas_call(
        _mm_body,
        grid=(N_BLOCKS,),
        in_specs=[
            pl.BlockSpec((R_PAIR, 2 * VE_DIM), lambda i: (i, 0)),
            pl.BlockSpec((VE_DIM, MODEL_DIM), lambda i: (0, 0)),
            pl.BlockSpec(memory_space=pltpu.SMEM),
        ],
        out_specs=pl.BlockSpec((2, R_PAIR, MODEL_DIM), lambda i: (0, i, 0)),
        out_shape=jax.ShapeDtypeStruct((2, HN, MODEL_DIM), jnp.float32),
    )(x, w_t, scale)


def kernel(token_ids, embed_weight, proj_weight, scale):
    ids = token_ids.astype(jnp.int32).reshape(-1)
    ids = (ids & ~(TBLK - 1)) + 2 * (ids & (HBLK - 1)) + ((ids >> 12) & 1)
    table_lin = _tc_transpose(embed_weight.T).reshape(VPAD, VE_DIM)
    paired = _sc_gather(ids, table_lin)
    out = _tc_project(paired, proj_weight.T, scale.reshape(1))
    return out.reshape(B, L, MODEL_DIM)


# speedup vs baseline: 2.3706x; 1.0176x over previous
import jax
import jax.numpy as jnp
from jax import lax
from jax.experimental import pallas as pl
from jax.experimental.pallas import tpu as pltpu
from jax.experimental.pallas import tpu_sc as plsc

VOCAB = 1000000
VE_DIM = 64
MODEL_DIM = 128
B = 4096
L = 200
N = B * L
HN = N // 2

NC = 2
NS = 16
NW = NC * NS
SPAN = HN // NW
CHUNK = 128
N_CHUNKS = SPAN // CHUNK
NBUF = 4

TBLK = 8192
HBLK = TBLK // 2
T_GRID = (VOCAB + TBLK - 1) // TBLK
VPAD = T_GRID * TBLK

R_PAIR = 8192
N_BLOCKS = HN // R_PAIR


def _tr_body(x_ref, o_ref):
    eye = jnp.eye(VE_DIM, dtype=jnp.float32)
    o_ref[:, :VE_DIM] = jnp.dot(
        x_ref[:, :HBLK].T, eye, preferred_element_type=jnp.float32
    )
    o_ref[:, VE_DIM:] = jnp.dot(
        x_ref[:, HBLK:].T, eye, preferred_element_type=jnp.float32
    )


def _tc_transpose(wt):
    return pl.pallas_call(
        _tr_body,
        grid=(T_GRID,),
        in_specs=[pl.BlockSpec((VE_DIM, TBLK), lambda i: (0, i))],
        out_specs=pl.BlockSpec((HBLK, 2 * VE_DIM), lambda i: (i, 0)),
        out_shape=jax.ShapeDtypeStruct((T_GRID * HBLK, 2 * VE_DIM), jnp.float32),
        compiler_params=pltpu.CompilerParams(fuse_transposed_lhs_in_matmul=True),
    )(wt)


def _sc_gather_body(idx_hbm, table_hbm, out_hbm, idx_v, bufa, bufb, sema, semb):
    c = lax.axis_index("c")
    s = lax.axis_index("s")
    wid = s * NC + c

    pltpu.sync_copy(idx_hbm.at[pl.ds(wid * SPAN, SPAN)], idx_v.at[0])
    pltpu.sync_copy(idx_hbm.at[pl.ds(HN + wid * SPAN, SPAN)], idx_v.at[1])

    def _fill(j, b):
        pltpu.async_copy(
            table_hbm.at[idx_v.at[0, pl.ds(j * CHUNK, CHUNK)]], bufa.at[b],
            sema.at[b],
        )
        pltpu.async_copy(
            table_hbm.at[idx_v.at[1, pl.ds(j * CHUNK, CHUNK)]], bufb.at[b],
            semb.at[b],
        )

    for b in range(NBUF):
        _fill(b, b)

    row0 = wid * SPAN

    def outer(j0, carry):
        for b in range(NBUF):
            j = j0 * NBUF + b
            pltpu.make_async_copy(
                table_hbm.at[idx_v.at[0, pl.ds(j * CHUNK, CHUNK)]], bufa.at[b],
                sema.at[b],
            ).wait()
            pltpu.make_async_copy(
                table_hbm.at[idx_v.at[1, pl.ds(j * CHUNK, CHUNK)]], bufb.at[b],
                semb.at[b],
            ).wait()
            r = row0 + j * CHUNK
            pltpu.sync_copy(bufa.at[b],
                            out_hbm.at[pl.ds(r, CHUNK), pl.ds(0, VE_DIM)])
            pltpu.sync_copy(bufb.at[b],
                            out_hbm.at[pl.ds(r, CHUNK), pl.ds(VE_DIM, VE_DIM)])

            @pl.when(j + NBUF < N_CHUNKS)
            def _refill(b=b, j=j):
                _fill(j + NBUF, b)

        return carry

    lax.fori_loop(0, N_CHUNKS // NBUF, outer, None)


def _sc_gather(idx, table):
    mesh = plsc.VectorSubcoreMesh(core_axis_name="c", subcore_axis_name="s")
    return pl.kernel(
        _sc_gather_body,
        out_type=jax.ShapeDtypeStruct((HN, 2 * VE_DIM), jnp.float32),
        name="sc_pair_gather",
        mesh=mesh,
        scratch_types=[
            pltpu.VMEM((2, SPAN), jnp.int32),
            pltpu.VMEM((NBUF, CHUNK, VE_DIM), jnp.float32),
            pltpu.VMEM((NBUF, CHUNK, VE_DIM), jnp.float32),
            pltpu.SemaphoreType.DMA((NBUF,)),
            pltpu.SemaphoreType.DMA((NBUF,)),
        ],
        compiler_params=pltpu.CompilerParams(use_tc_tiling_on_sc=False),
    )(idx, table)


def _mm_body(x_ref, w_ref, s_ref, o_ref):
    sc = s_ref[0]
    w = w_ref[...]
    o_ref[0] = (
        jnp.dot(x_ref[:, :VE_DIM], w, preferred_element_type=jnp.float32) * sc
    )
    o_ref[1] = (
        jnp.dot(x_ref[:, VE_DIM:], w, preferred_element_type=jnp.float32) * sc
    )


def _tc_project(x, w_t, scale):
    return pl.pallas_call(
        _mm_body,
        grid=(N_BLOCKS,),
        in_specs=[
            pl.BlockSpec((R_PAIR, 2 * VE_DIM), lambda i: (i, 0)),
            pl.BlockSpec((VE_DIM, MODEL_DIM), lambda i: (0, 0)),
            pl.BlockSpec(memory_space=pltpu.SMEM),
        ],
        out_specs=pl.BlockSpec((2, R_PAIR, MODEL_DIM), lambda i: (0, i, 0)),
        out_shape=jax.ShapeDtypeStruct((2, HN, MODEL_DIM), jnp.float32),
    )(x, w_t, scale)


def kernel(token_ids, embed_weight, proj_weight, scale):
    ids = token_ids.astype(jnp.int32).reshape(-1)
    ids = (ids & ~(TBLK - 1)) + 2 * (ids & (HBLK - 1)) + ((ids >> 12) & 1)
    table_lin = _tc_transpose(embed_weight.T).reshape(VPAD, VE_DIM)
    paired = _sc_gather(ids, table_lin)
    out = _tc_project(paired, proj_weight.T, scale.reshape(1))
    return out.reshape(B, L, MODEL_DIM)
